# baseline (speedup 1.0000x reference)
import functools

import jax
import jax.numpy as jnp
from jax import lax
from jax.experimental import pallas as pl
from jax.experimental.pallas import tpu as pltpu
from jax.experimental.pallas import tpu_sc as plsc

_N = 10000
_D = 256
_HD = 128
_O = 128
_E = 160000
_NC = 2
_NS = 16
_ESUB = _E // _NS
_CHUNK = 100
_NCHUNK = _ESUB // _CHUNK
_NPAD = 10240
_RSUB = _NPAD // _NS
_F32 = jnp.float32
_HP = lax.Precision.HIGHEST


def _seg_sum_body(compute_counts, *refs):
    if compute_counts:
        (xl, xr, edges, zrows, zcnt, ones_h,
         agg_o, cnt_o,
         rows0, rows1, idx0, idx1, idx2, idx3, onesv, acc, cntacc,
         g, s0, s1, t0, t1, i0, i1, i2, i3) = refs
        tsem_ = (t0, t1)
    else:
        (xl, xr, edges, zrows,
         agg_o,
         rows0, rows1, idx0, idx1, idx2, idx3, acc,
         g, s0, s1, i0, i1, i2, i3) = refs
        tsem_ = None

    rows_ = (rows0, rows1)
    idx_ = (idx0, idx1, idx2, idx3)
    isem_ = (i0, i1, i2, i3)
    ssem_ = (s0, s1)

    cid = lax.axis_index("c")
    sid = lax.axis_index("s")
    row0 = sid * _RSUB

    pltpu.sync_copy(zrows.at[pl.ds(row0, _RSUB)], acc.at[pl.ds(row0, _RSUB)])
    if compute_counts:
        @pl.when(cid == 0)
        def _():
            pltpu.sync_copy(zcnt.at[pl.ds(row0, _RSUB)],
                            cntacc.at[pl.ds(row0, _RSUB)])
            pltpu.sync_copy(ones_h, onesv)
    plsc.subcore_barrier()

    def run(x_hbm, with_counts):
        def prefetch(c, q):
            pltpu.async_copy(edges.at[sid, c], idx_[q], isem_[q])

        def wait_idx(q):
            pltpu.make_async_copy(edges.at[sid, 0], idx_[q], isem_[q]).wait()

        def gather(q, b):
            pltpu.async_copy(x_hbm.at[idx_[q].at[0]], rows_[b], g).wait()

        def scatter(q, b):
            pltpu.async_copy(rows_[b], acc.at[idx_[q].at[1]], ssem_[b],
                             add=True)
            if with_counts:
                pltpu.async_copy(onesv, cntacc.at[idx_[q].at[1]], tsem_[b],
                                 add=True)

        def wait_scatter(b):
            pltpu.make_async_copy(rows_[b], acc.at[idx_[0].at[1]],
                                  ssem_[b]).wait()
            if with_counts:
                pltpu.make_async_copy(onesv, cntacc.at[idx_[0].at[1]],
                                      tsem_[b]).wait()

        prefetch(0, 0)
        prefetch(1, 1)
        wait_idx(0)
        gather(0, 0)
        scatter(0, 0)
        prefetch(2, 2)
        wait_idx(1)
        gather(1, 1)
        scatter(1, 1)
        prefetch(3, 3)

        @pl.loop(0, (_NCHUNK - 4) // 4)
        def _(k):
            c0 = 2 + 4 * k
            for i, (q, b) in enumerate(((2, 0), (3, 1), (0, 0), (1, 1))):
                wait_scatter(b)
                wait_idx(q)
                gather(q, b)
                scatter(q, b)
                prefetch(c0 + i + 2, (q + 2) % 4)

        wait_scatter(0)
        wait_idx(2)
        gather(2, 0)
        scatter(2, 0)
        wait_scatter(1)
        wait_idx(3)
        gather(3, 1)
        scatter(3, 1)
        wait_scatter(0)
        wait_scatter(1)

    @pl.when(cid == 0)
    def _():
        run(xl, compute_counts)

    @pl.when(cid == 1)
    def _():
        run(xr, False)

    plsc.subcore_barrier()
    pltpu.sync_copy(acc.at[pl.ds(row0, _RSUB)],
                    agg_o.at[cid, pl.ds(row0, _RSUB)])
    if compute_counts:
        @pl.when(cid == 0)
        def _():
            pltpu.sync_copy(cntacc.at[pl.ds(row0, _RSUB)],
                            cnt_o.at[pl.ds(row0, _RSUB)])


def _make_seg_sum(compute_counts):
    mesh = plsc.VectorSubcoreMesh(core_axis_name="c", subcore_axis_name="s")
    out_type = [jax.ShapeDtypeStruct((_NC, _NPAD, _HD), _F32)]
    scratch = [
        pltpu.VMEM((_CHUNK, _HD), _F32),
        pltpu.VMEM((_CHUNK, _HD), _F32),
        pltpu.VMEM((2, _CHUNK), jnp.int32),
        pltpu.VMEM((2, _CHUNK), jnp.int32),
        pltpu.VMEM((2, _CHUNK), jnp.int32),
        pltpu.VMEM((2, _CHUNK), jnp.int32),
    ]
    if compute_counts:
        out_type.append(jax.ShapeDtypeStruct((_NPAD, 16), _F32))
        scratch.append(pltpu.VMEM((_CHUNK, 16), _F32))
    scratch.append(pltpu.VMEM_SHARED((_NPAD, _HD), _F32))
    if compute_counts:
        scratch.append(pltpu.VMEM_SHARED((_NPAD, 16), _F32))
    nsem = 9 if compute_counts else 7
    scratch.extend([pltpu.SemaphoreType.DMA] * nsem)
    return pl.kernel(
        functools.partial(_seg_sum_body, compute_counts),
        out_type=out_type,
        mesh=mesh,
        scratch_types=scratch,
        compiler_params=pltpu.CompilerParams(use_tc_tiling_on_sc=False),
    )


_seg_sum_cnt = _make_seg_sum(True)
_seg_sum_nocnt = _make_seg_sum(False)


_R = 2000


def _dot(a, b):
    return jnp.dot(a, b, precision=_HP, preferred_element_type=_F32)


def _pre_body(xs, xu, w1r, b1, w2r, b2, pre1_o, pre2_o):
    pre1_o[...] = _dot(xs[...], w1r[...]) + b1[...]
    pre2_o[...] = _dot(xu[...], w2r[...]) + b2[...]


def _mid_item_body(agg1, cnt1, pre1, w1l, item0_o, item1_o):
    r1 = 1.0 / jnp.maximum(cnt1[:, 0:1], 1.0)
    w1l_m = w1l[...]
    item = (_dot(agg1[0] * r1, w1l_m[:_HD])
            + _dot(agg1[1] * r1, w1l_m[_HD:])
            + pre1[...])
    item = jnp.maximum(item, 0.0)
    item0_o[...] = item[:, :_HD]
    item1_o[...] = item[:, _HD:]


def _mid_user_body(agg2, cnt2, pre2, w2l, w3r, b3, u2r_o):
    r2 = 1.0 / jnp.maximum(cnt2[:, 0:1], 1.0)
    w2l_m = w2l[...]
    user2 = (_dot(agg2[0] * r2, w2l_m[:_HD])
             + _dot(agg2[1] * r2, w2l_m[_HD:])
             + pre2[...])
    user2 = jnp.maximum(user2, 0.0)
    u2r_o[...] = _dot(user2, w3r[...]) + b3[...]


def _fin_body(agg3, cnt2, u2r, w3l, wlin, blin, out_o):
    r = 1.0 / jnp.maximum(cnt2[:, 0:1], 1.0)
    w3l_m = w3l[...]
    u3 = (_dot(agg3[0] * r, w3l_m[:_HD])
          + _dot(agg3[1] * r, w3l_m[_HD:])
          + u2r[...])
    u3 = jnp.maximum(u3, 0.0)
    out_o[...] = _dot(u3, wlin[...]) + blin[...]


def _full(shape):
    return pl.BlockSpec(shape, lambda i: tuple(0 for _ in shape))


_rows3 = pl.BlockSpec((_NC, _R, _HD), lambda i: (0, i, 0))
_rowsD = pl.BlockSpec((_R, _D), lambda i: (i, 0))
_rowsC = pl.BlockSpec((_R, 16), lambda i: (i, 0))
_rowsH = pl.BlockSpec((_R, _HD), lambda i: (i, 0))

_pre = pl.pallas_call(
    _pre_body,
    grid=(_N // _R,),
    in_specs=[_rowsD, _rowsD,
              _full((_D, _D)), _full((1, _D)),
              _full((_D, _D)), _full((1, _D))],
    out_specs=[_rowsD, _rowsD],
    out_shape=[jax.ShapeDtypeStruct((_N, _D), _F32),
               jax.ShapeDtypeStruct((_N, _D), _F32)],
)

_mid_item = pl.pallas_call(
    _mid_item_body,
    grid=(_N // _R,),
    in_specs=[_rows3, _rowsC, _rowsD, _full((_D, _D))],
    out_specs=[_rowsH, _rowsH],
    out_shape=[jax.ShapeDtypeStruct((_N, _HD), _F32),
               jax.ShapeDtypeStruct((_N, _HD), _F32)],
)

_mid_user = pl.pallas_call(
    _mid_user_body,
    grid=(_N // _R,),
    in_specs=[_rows3, _rowsC, _rowsD,
              _full((_D, _D)), _full((_D, _D)), _full((1, _D))],
    out_specs=_rowsD,
    out_shape=jax.ShapeDtypeStruct((_N, _D), _F32),
)

_fin = pl.pallas_call(
    _fin_body,
    grid=(_N // _R,),
    in_specs=[_rows3, _rowsC, _rowsD,
              _full((_D, _D)), _full((_D, _O)), _full((1, _O))],
    out_specs=pl.BlockSpec((_R, _O), lambda i: (i, 0)),
    out_shape=jax.ShapeDtypeStruct((_N, _O), _F32),
)


def kernel(x_store, x_user, edge_index_store_store, edge_index_store_user,
           W1_l, b1, W1_r, W2_l, b2, W2_r, W3_l, b3, W3_r, W_lin, b_lin):
    xsl = x_store[:, :_HD]
    xsr = x_store[:, _HD:]
    ess = jnp.transpose(
        edge_index_store_store.reshape(2, _NS, _NCHUNK, _CHUNK),
        (1, 2, 0, 3))
    esu = jnp.transpose(
        edge_index_store_user.reshape(2, _NS, _NCHUNK, _CHUNK),
        (1, 2, 0, 3))
    zrows = jnp.zeros((_NPAD, _HD), _F32)
    zcnt = jnp.zeros((_NPAD, 16), _F32)
    ones = jnp.ones((_CHUNK, 16), _F32)

    pre1, pre2 = _pre(x_store, x_user,
                      W1_r, b1.reshape(1, _D),
                      W2_r, b2.reshape(1, _D))
    agg1, cnt1 = _seg_sum_cnt(xsl, xsr, ess, zrows, zcnt, ones)
    agg2, cnt2 = _seg_sum_cnt(xsl, xsr, esu, zrows, zcnt, ones)
    item0, item1 = _mid_item(agg1, cnt1, pre1, W1_l)
    u2r = _mid_user(agg2, cnt2, pre2, W2_l, W3_r, b3.reshape(1, _D))
    (agg3,) = _seg_sum_nocnt(item0, item1, esu, zrows)
    return _fin(agg3, cnt2, u2r, W3_l, W_lin, b_lin.reshape(1, _O))


# speedup vs baseline: 1.1489x; 1.1489x over previous
import functools

import jax
import jax.numpy as jnp
from jax import lax
from jax.experimental import pallas as pl
from jax.experimental.pallas import tpu as pltpu
from jax.experimental.pallas import tpu_sc as plsc

_N = 10000
_D = 256
_HD = 128
_O = 128
_E = 160000
_NC = 2
_NS = 16
_ESUB = _E // _NS
_CHUNK = 125
_NCHUNK = _ESUB // _CHUNK
_CHUNK3 = 100
_NCHUNK3 = _ESUB // _CHUNK3
_NSUP = _NCHUNK3 // 4
_NPAD = 10240
_RSUB = _NPAD // _NS
_F32 = jnp.float32
_HP = lax.Precision.HIGHEST


def _seg_sum_body(compute_counts, *refs):
    if compute_counts:
        (xl, xr, edges, zrows, zcnt, ones_h,
         agg_o, cnt_o,
         rows0, rows1, idx0, idx1, idx2, idx3, onesv, acc, cntacc,
         g, s0, s1, t0, t1, i0, i1, i2, i3) = refs
        tsem_ = (t0, t1)
    else:
        (xl, xr, edges, zrows,
         agg_o,
         rows0, rows1, idx0, idx1, idx2, idx3, acc,
         g, s0, s1, i0, i1, i2, i3) = refs
        tsem_ = None

    rows_ = (rows0, rows1)
    idx_ = (idx0, idx1, idx2, idx3)
    isem_ = (i0, i1, i2, i3)
    ssem_ = (s0, s1)

    cid = lax.axis_index("c")
    sid = lax.axis_index("s")
    row0 = sid * _RSUB

    pltpu.sync_copy(zrows.at[pl.ds(row0, _RSUB)], acc.at[pl.ds(row0, _RSUB)])
    if compute_counts:
        @pl.when(cid == 0)
        def _():
            pltpu.sync_copy(zcnt.at[pl.ds(row0, _RSUB)],
                            cntacc.at[pl.ds(row0, _RSUB)])
            pltpu.sync_copy(ones_h, onesv)
    plsc.subcore_barrier()

    def run(x_hbm, with_counts):
        def prefetch(c, q):
            pltpu.async_copy(edges.at[sid, c], idx_[q], isem_[q])

        def wait_idx(q):
            pltpu.make_async_copy(edges.at[sid, 0], idx_[q], isem_[q]).wait()

        def gather(q, b):
            pltpu.async_copy(x_hbm.at[idx_[q].at[0]], rows_[b], g).wait()

        def scatter(q, b):
            pltpu.async_copy(rows_[b], acc.at[idx_[q].at[1]], ssem_[b],
                             add=True)
            if with_counts:
                pltpu.async_copy(onesv, cntacc.at[idx_[q].at[1]], tsem_[b],
                                 add=True)

        def wait_scatter(b):
            pltpu.make_async_copy(rows_[b], acc.at[idx_[0].at[1]],
                                  ssem_[b]).wait()
            if with_counts:
                pltpu.make_async_copy(onesv, cntacc.at[idx_[0].at[1]],
                                      tsem_[b]).wait()

        prefetch(0, 0)
        prefetch(1, 1)
        wait_idx(0)
        gather(0, 0)
        scatter(0, 0)
        prefetch(2, 2)
        wait_idx(1)
        gather(1, 1)
        scatter(1, 1)
        prefetch(3, 3)

        @pl.loop(0, (_NCHUNK - 4) // 4)
        def _(k):
            c0 = 2 + 4 * k
            for i, (q, b) in enumerate(((2, 0), (3, 1), (0, 0), (1, 1))):
                wait_scatter(b)
                wait_idx(q)
                gather(q, b)
                scatter(q, b)
                prefetch(c0 + i + 2, (q + 2) % 4)

        wait_scatter(0)
        wait_idx(2)
        gather(2, 0)
        scatter(2, 0)
        wait_scatter(1)
        wait_idx(3)
        gather(3, 1)
        scatter(3, 1)
        wait_scatter(0)
        wait_scatter(1)

    @pl.when(cid == 0)
    def _():
        run(xl, compute_counts)

    @pl.when(cid == 1)
    def _():
        run(xr, False)

    plsc.subcore_barrier()
    pltpu.sync_copy(acc.at[pl.ds(row0, _RSUB)],
                    agg_o.at[cid, pl.ds(row0, _RSUB)])
    if compute_counts:
        @pl.when(cid == 0)
        def _():
            pltpu.sync_copy(cntacc.at[pl.ds(row0, _RSUB)],
                            cnt_o.at[pl.ds(row0, _RSUB)])


def _seg_sum3_body(with_counts, *refs):
    if with_counts:
        (xl, xr, edges, zrows, zcnt, ones_h,
         agg_o, cnt_o,
         r0, r1, r2, sp0, sp1, sp2, onesv, acc, cntacc,
         g0, g1, g2, s0, s1, s2, t0, t1, t2, i0, i1, i2) = refs
        t_ = (t0, t1, t2)
    else:
        (xl, xr, edges, zrows,
         agg_o,
         r0, r1, r2, sp0, sp1, sp2, acc,
         g0, g1, g2, s0, s1, s2, i0, i1, i2) = refs
        t_ = None
    rows_ = (r0, r1, r2)
    sup_ = (sp0, sp1, sp2)
    g_ = (g0, g1, g2)
    s_ = (s0, s1, s2)
    i_ = (i0, i1, i2)

    cid = lax.axis_index("c")
    sid = lax.axis_index("s")
    row0 = sid * _RSUB

    pltpu.sync_copy(zrows.at[pl.ds(row0, _RSUB)], acc.at[pl.ds(row0, _RSUB)])
    if with_counts:
        @pl.when(cid == 0)
        def _():
            pltpu.sync_copy(zcnt.at[pl.ds(row0, _RSUB)],
                            cntacc.at[pl.ds(row0, _RSUB)])
            pltpu.sync_copy(ones_h, onesv)
    plsc.subcore_barrier()

    def run(x_hbm, wc):
        def prefetch(s_dyn, slot):
            pltpu.async_copy(edges.at[sid, s_dyn], sup_[slot], i_[slot])

        def wait_sup(slot):
            pltpu.make_async_copy(edges.at[sid, 0], sup_[slot],
                                  i_[slot]).wait()

        def gather(r, slot, b):
            pltpu.async_copy(x_hbm.at[sup_[slot].at[r, 0]], rows_[b], g_[b])

        def wait_g(b):
            pltpu.make_async_copy(x_hbm.at[sup_[0].at[0, 0]], rows_[b],
                                  g_[b]).wait()

        def scatter(r, slot, b):
            pltpu.async_copy(rows_[b], acc.at[sup_[slot].at[r, 1]], s_[b],
                             add=True)
            if wc:
                pltpu.async_copy(onesv, cntacc.at[sup_[slot].at[r, 1]],
                                 t_[b], add=True)

        def wait_s(b):
            pltpu.make_async_copy(rows_[b], acc.at[sup_[0].at[0, 1]],
                                  s_[b]).wait()
            if wc:
                pltpu.make_async_copy(onesv, cntacc.at[sup_[0].at[0, 1]],
                                      t_[b]).wait()

        prefetch(0, 0)
        prefetch(1, 1)
        wait_sup(0)
        gather(0, 0, 0)
        gather(1, 0, 1)
        wait_g(0)
        scatter(0, 0, 0)
        gather(2, 0, 2)
        wait_g(1)
        scatter(1, 0, 1)
        wait_s(0)
        gather(3, 0, 0)

        ngroups = (_NCHUNK3 - 2 - 12) // 12
        @pl.loop(0, ngroups)
        def _(k):
            for cc in range(12):
                b = (2 + cc) % 3
                wait_g(b)
                scatter((2 + cc) % 4, ((2 + cc) // 4) % 3, b)
                wait_s((4 + cc) % 3)
                if cc % 4 == 0:
                    prefetch(3 * k + (cc + 8) // 4, ((cc + 8) // 4) % 3)
                    wait_sup(((cc + 4) // 4) % 3)
                gather(cc % 4, ((cc + 4) // 4) % 3, (4 + cc) % 3)

        for c in range(2 + 12 * ngroups, _NCHUNK3):
            wait_g(c % 3)
            scatter(c % 4, (c // 4) % 3, c % 3)
            wait_s((c + 2) % 3)
            if (c + 6) % 4 == 0 and (c + 6) // 4 < _NSUP:
                prefetch((c + 6) // 4, ((c + 6) // 4) % 3)
            if c + 2 < _NCHUNK3:
                if (c + 2) % 4 == 0:
                    wait_sup(((c + 2) // 4) % 3)
                gather((c + 2) % 4, ((c + 2) // 4) % 3, (c + 2) % 3)
        wait_s((_NCHUNK3 - 1) % 3)

    @pl.when(cid == 0)
    def _():
        run(xl, with_counts)

    @pl.when(cid == 1)
    def _():
        run(xr, False)

    plsc.subcore_barrier()
    pltpu.sync_copy(acc.at[pl.ds(row0, _RSUB)],
                    agg_o.at[cid, pl.ds(row0, _RSUB)])
    if with_counts:
        @pl.when(cid == 0)
        def _():
            pltpu.sync_copy(cntacc.at[pl.ds(row0, _RSUB)],
                            cnt_o.at[pl.ds(row0, _RSUB)])


def _make_seg_sum3(with_counts):
    mesh = plsc.VectorSubcoreMesh(core_axis_name="c", subcore_axis_name="s")
    out_type = [jax.ShapeDtypeStruct((_NC, _NPAD, _HD), _F32)]
    scratch = [
        pltpu.VMEM((_CHUNK3, _HD), _F32),
        pltpu.VMEM((_CHUNK3, _HD), _F32),
        pltpu.VMEM((_CHUNK3, _HD), _F32),
        pltpu.VMEM((4, 2, _CHUNK3), jnp.int32),
        pltpu.VMEM((4, 2, _CHUNK3), jnp.int32),
        pltpu.VMEM((4, 2, _CHUNK3), jnp.int32),
    ]
    if with_counts:
        out_type.append(jax.ShapeDtypeStruct((_NPAD, 16), _F32))
        scratch.append(pltpu.VMEM((_CHUNK3, 16), _F32))
    scratch.append(pltpu.VMEM_SHARED((_NPAD, _HD), _F32))
    if with_counts:
        scratch.append(pltpu.VMEM_SHARED((_NPAD, 16), _F32))
    nsem = 12 if with_counts else 9
    scratch.extend([pltpu.SemaphoreType.DMA] * nsem)
    return pl.kernel(
        functools.partial(_seg_sum3_body, with_counts),
        out_type=out_type,
        mesh=mesh,
        scratch_types=scratch,
        compiler_params=pltpu.CompilerParams(use_tc_tiling_on_sc=False),
    )


def _make_seg_sum(compute_counts):
    mesh = plsc.VectorSubcoreMesh(core_axis_name="c", subcore_axis_name="s")
    out_type = [jax.ShapeDtypeStruct((_NC, _NPAD, _HD), _F32)]
    scratch = [
        pltpu.VMEM((_CHUNK, _HD), _F32),
        pltpu.VMEM((_CHUNK, _HD), _F32),
        pltpu.VMEM((2, _CHUNK), jnp.int32),
        pltpu.VMEM((2, _CHUNK), jnp.int32),
        pltpu.VMEM((2, _CHUNK), jnp.int32),
        pltpu.VMEM((2, _CHUNK), jnp.int32),
    ]
    if compute_counts:
        out_type.append(jax.ShapeDtypeStruct((_NPAD, 16), _F32))
        scratch.append(pltpu.VMEM((_CHUNK, 16), _F32))
    scratch.append(pltpu.VMEM_SHARED((_NPAD, _HD), _F32))
    if compute_counts:
        scratch.append(pltpu.VMEM_SHARED((_NPAD, 16), _F32))
    nsem = 9 if compute_counts else 7
    scratch.extend([pltpu.SemaphoreType.DMA] * nsem)
    return pl.kernel(
        functools.partial(_seg_sum_body, compute_counts),
        out_type=out_type,
        mesh=mesh,
        scratch_types=scratch,
        compiler_params=pltpu.CompilerParams(use_tc_tiling_on_sc=False),
    )


_seg_sum_cnt = _make_seg_sum(True)
_seg_sum_nocnt = _make_seg_sum3(False)


_R = 2000


def _dot(a, b):
    return jnp.dot(a, b, precision=_HP, preferred_element_type=_F32)


def _pre_body(xs, xu, w1r, b1, w2r, b2, pre1_o, pre2_o):
    pre1_o[...] = _dot(xs[...], w1r[...]) + b1[...]
    pre2_o[...] = _dot(xu[...], w2r[...]) + b2[...]


def _mid_item_body(agg1, cnt1, pre1, w1l, item0_o, item1_o):
    r1 = 1.0 / jnp.maximum(cnt1[:, 0:1], 1.0)
    w1l_m = w1l[...]
    item = (_dot(agg1[0] * r1, w1l_m[:_HD])
            + _dot(agg1[1] * r1, w1l_m[_HD:])
            + pre1[...])
    item = jnp.maximum(item, 0.0)
    item0_o[...] = item[:, :_HD]
    item1_o[...] = item[:, _HD:]


def _mid_user_body(agg2, cnt2, pre2, w2l, w3r, b3, u2r_o):
    r2 = 1.0 / jnp.maximum(cnt2[:, 0:1], 1.0)
    w2l_m = w2l[...]
    user2 = (_dot(agg2[0] * r2, w2l_m[:_HD])
             + _dot(agg2[1] * r2, w2l_m[_HD:])
             + pre2[...])
    user2 = jnp.maximum(user2, 0.0)
    u2r_o[...] = _dot(user2, w3r[...]) + b3[...]


def _fin_body(agg3, cnt2, u2r, w3l, wlin, blin, out_o):
    r = 1.0 / jnp.maximum(cnt2[:, 0:1], 1.0)
    w3l_m = w3l[...]
    u3 = (_dot(agg3[0] * r, w3l_m[:_HD])
          + _dot(agg3[1] * r, w3l_m[_HD:])
          + u2r[...])
    u3 = jnp.maximum(u3, 0.0)
    out_o[...] = _dot(u3, wlin[...]) + blin[...]


def _full(shape):
    return pl.BlockSpec(shape, lambda i: tuple(0 for _ in shape))


_rows3 = pl.BlockSpec((_NC, _R, _HD), lambda i: (0, i, 0))
_rowsD = pl.BlockSpec((_R, _D), lambda i: (i, 0))
_rowsC = pl.BlockSpec((_R, 16), lambda i: (i, 0))
_rowsH = pl.BlockSpec((_R, _HD), lambda i: (i, 0))

_pre = pl.pallas_call(
    _pre_body,
    grid=(_N // _R,),
    in_specs=[_rowsD, _rowsD,
              _full((_D, _D)), _full((1, _D)),
              _full((_D, _D)), _full((1, _D))],
    out_specs=[_rowsD, _rowsD],
    out_shape=[jax.ShapeDtypeStruct((_N, _D), _F32),
               jax.ShapeDtypeStruct((_N, _D), _F32)],
)

_mid_item = pl.pallas_call(
    _mid_item_body,
    grid=(_N // _R,),
    in_specs=[_rows3, _rowsC, _rowsD, _full((_D, _D))],
    out_specs=[_rowsH, _rowsH],
    out_shape=[jax.ShapeDtypeStruct((_N, _HD), _F32),
               jax.ShapeDtypeStruct((_N, _HD), _F32)],
)

_mid_user = pl.pallas_call(
    _mid_user_body,
    grid=(_N // _R,),
    in_specs=[_rows3, _rowsC, _rowsD,
              _full((_D, _D)), _full((_D, _D)), _full((1, _D))],
    out_specs=_rowsD,
    out_shape=jax.ShapeDtypeStruct((_N, _D), _F32),
)

_fin = pl.pallas_call(
    _fin_body,
    grid=(_N // _R,),
    in_specs=[_rows3, _rowsC, _rowsD,
              _full((_D, _D)), _full((_D, _O)), _full((1, _O))],
    out_specs=pl.BlockSpec((_R, _O), lambda i: (i, 0)),
    out_shape=jax.ShapeDtypeStruct((_N, _O), _F32),
)


def kernel(x_store, x_user, edge_index_store_store, edge_index_store_user,
           W1_l, b1, W1_r, W2_l, b2, W2_r, W3_l, b3, W3_r, W_lin, b_lin):
    xsl = x_store[:, :_HD]
    xsr = x_store[:, _HD:]
    ess = jnp.transpose(
        edge_index_store_store.reshape(2, _NS, _NCHUNK, _CHUNK),
        (1, 2, 0, 3))
    esu = jnp.transpose(
        edge_index_store_user.reshape(2, _NS, _NCHUNK, _CHUNK),
        (1, 2, 0, 3))
    esu3 = jnp.transpose(
        edge_index_store_user.reshape(2, _NS, _NSUP, 4, _CHUNK3),
        (1, 2, 3, 0, 4))
    zrows = jnp.zeros((_NPAD, _HD), _F32)
    zcnt = jnp.zeros((_NPAD, 16), _F32)
    ones = jnp.ones((_CHUNK, 16), _F32)

    pre1, pre2 = _pre(x_store, x_user,
                      W1_r, b1.reshape(1, _D),
                      W2_r, b2.reshape(1, _D))
    agg1, cnt1 = _seg_sum_cnt(xsl, xsr, ess, zrows, zcnt, ones)
    agg2, cnt2 = _seg_sum_cnt(xsl, xsr, esu, zrows, zcnt, ones)
    item0, item1 = _mid_item(agg1, cnt1, pre1, W1_l)
    u2r = _mid_user(agg2, cnt2, pre2, W2_l, W3_r, b3.reshape(1, _D))
    (agg3,) = _seg_sum_nocnt(item0, item1, esu3, zrows)
    return _fin(agg3, cnt2, u2r, W3_l, W_lin, b_lin.reshape(1, _O))


# speedup vs baseline: 1.2689x; 1.1044x over previous
import functools

import jax
import jax.numpy as jnp
from jax import lax
from jax.experimental import pallas as pl
from jax.experimental.pallas import tpu as pltpu
from jax.experimental.pallas import tpu_sc as plsc

_N = 10000
_D = 256
_HD = 128
_O = 128
_E = 160000
_NC = 2
_NS = 16
_ESUB = _E // _NS
_CHUNK = 125
_NCHUNK = _ESUB // _CHUNK
_CHUNK3 = 100
_NCHUNK3 = _ESUB // _CHUNK3
_NSUP = _NCHUNK3 // 4
_NPAD = 10240
_RSUB = _NPAD // _NS
_F32 = jnp.float32
_HP = lax.Precision.HIGHEST


def _seg_sum3_body(with_counts, *refs):
    if with_counts:
        (xl, xr, edges, zrows, zcnt, ones_h,
         agg_o, cnt_o,
         r0, r1, r2, sp0, sp1, sp2, onesv, acc, cntacc,
         g0, g1, g2, s0, s1, s2, t0, t1, t2, i0, i1, i2) = refs
        t_ = (t0, t1, t2)
    else:
        (xl, xr, edges, zrows,
         agg_o,
         r0, r1, r2, sp0, sp1, sp2, acc,
         g0, g1, g2, s0, s1, s2, i0, i1, i2) = refs
        t_ = None
    rows_ = (r0, r1, r2)
    sup_ = (sp0, sp1, sp2)
    g_ = (g0, g1, g2)
    s_ = (s0, s1, s2)
    i_ = (i0, i1, i2)

    cid = lax.axis_index("c")
    sid = lax.axis_index("s")
    row0 = sid * _RSUB

    pltpu.sync_copy(zrows.at[pl.ds(row0, _RSUB)], acc.at[pl.ds(row0, _RSUB)])
    if with_counts:
        @pl.when(cid == 0)
        def _():
            pltpu.sync_copy(zcnt.at[pl.ds(row0, _RSUB)],
                            cntacc.at[pl.ds(row0, _RSUB)])
            pltpu.sync_copy(ones_h, onesv)
    plsc.subcore_barrier()

    def run(x_hbm, wc):
        def prefetch(s_dyn, slot):
            pltpu.async_copy(edges.at[sid, s_dyn], sup_[slot], i_[slot])

        def wait_sup(slot):
            pltpu.make_async_copy(edges.at[sid, 0], sup_[slot],
                                  i_[slot]).wait()

        def gather(r, slot, b):
            pltpu.async_copy(x_hbm.at[sup_[slot].at[r, 0]], rows_[b], g_[b])

        def wait_g(b):
            pltpu.make_async_copy(x_hbm.at[sup_[0].at[0, 0]], rows_[b],
                                  g_[b]).wait()

        def scatter(r, slot, b):
            pltpu.async_copy(rows_[b], acc.at[sup_[slot].at[r, 1]], s_[b],
                             add=True)
            if wc:
                pltpu.async_copy(onesv, cntacc.at[sup_[slot].at[r, 1]],
                                 t_[b], add=True)

        def wait_s(b):
            pltpu.make_async_copy(rows_[b], acc.at[sup_[0].at[0, 1]],
                                  s_[b]).wait()
            if wc:
                pltpu.make_async_copy(onesv, cntacc.at[sup_[0].at[0, 1]],
                                      t_[b]).wait()

        prefetch(0, 0)
        prefetch(1, 1)
        wait_sup(0)
        gather(0, 0, 0)
        gather(1, 0, 1)
        wait_g(0)
        scatter(0, 0, 0)
        gather(2, 0, 2)
        wait_g(1)
        scatter(1, 0, 1)
        wait_s(0)
        gather(3, 0, 0)

        ngroups = (_NCHUNK3 - 2 - 12) // 12
        @pl.loop(0, ngroups)
        def _(k):
            for cc in range(12):
                b = (2 + cc) % 3
                wait_g(b)
                scatter((2 + cc) % 4, ((2 + cc) // 4) % 3, b)
                wait_s((4 + cc) % 3)
                if cc % 4 == 0:
                    prefetch(3 * k + (cc + 8) // 4, ((cc + 8) // 4) % 3)
                    wait_sup(((cc + 4) // 4) % 3)
                gather(cc % 4, ((cc + 4) // 4) % 3, (4 + cc) % 3)

        for c in range(2 + 12 * ngroups, _NCHUNK3):
            wait_g(c % 3)
            scatter(c % 4, (c // 4) % 3, c % 3)
            wait_s((c + 2) % 3)
            if (c + 6) % 4 == 0 and (c + 6) // 4 < _NSUP:
                prefetch((c + 6) // 4, ((c + 6) // 4) % 3)
            if c + 2 < _NCHUNK3:
                if (c + 2) % 4 == 0:
                    wait_sup(((c + 2) // 4) % 3)
                gather((c + 2) % 4, ((c + 2) // 4) % 3, (c + 2) % 3)
        wait_s((_NCHUNK3 - 1) % 3)

    @pl.when(cid == 0)
    def _():
        run(xl, with_counts)

    @pl.when(cid == 1)
    def _():
        run(xr, False)

    plsc.subcore_barrier()
    pltpu.sync_copy(acc.at[pl.ds(row0, _RSUB)],
                    agg_o.at[cid, pl.ds(row0, _RSUB)])
    if with_counts:
        @pl.when(cid == 0)
        def _():
            pltpu.sync_copy(cntacc.at[pl.ds(row0, _RSUB)],
                            cnt_o.at[pl.ds(row0, _RSUB)])


def _make_seg_sum3(with_counts):
    mesh = plsc.VectorSubcoreMesh(core_axis_name="c", subcore_axis_name="s")
    out_type = [jax.ShapeDtypeStruct((_NC, _NPAD, _HD), _F32)]
    scratch = [
        pltpu.VMEM((_CHUNK3, _HD), _F32),
        pltpu.VMEM((_CHUNK3, _HD), _F32),
        pltpu.VMEM((_CHUNK3, _HD), _F32),
        pltpu.VMEM((4, 2, _CHUNK3), jnp.int32),
        pltpu.VMEM((4, 2, _CHUNK3), jnp.int32),
        pltpu.VMEM((4, 2, _CHUNK3), jnp.int32),
    ]
    if with_counts:
        out_type.append(jax.ShapeDtypeStruct((_NPAD, 16), _F32))
        scratch.append(pltpu.VMEM((_CHUNK3, 16), _F32))
    scratch.append(pltpu.VMEM_SHARED((_NPAD, _HD), _F32))
    if with_counts:
        scratch.append(pltpu.VMEM_SHARED((_NPAD, 16), _F32))
    nsem = 12 if with_counts else 9
    scratch.extend([pltpu.SemaphoreType.DMA] * nsem)
    return pl.kernel(
        functools.partial(_seg_sum3_body, with_counts),
        out_type=out_type,
        mesh=mesh,
        scratch_types=scratch,
        compiler_params=pltpu.CompilerParams(use_tc_tiling_on_sc=False),
    )


def _make_seg_sum(compute_counts):
    mesh = plsc.VectorSubcoreMesh(core_axis_name="c", subcore_axis_name="s")
    out_type = [jax.ShapeDtypeStruct((_NC, _NPAD, _HD), _F32)]
    scratch = [
        pltpu.VMEM((_CHUNK, _HD), _F32),
        pltpu.VMEM((_CHUNK, _HD), _F32),
        pltpu.VMEM((2, _CHUNK), jnp.int32),
        pltpu.VMEM((2, _CHUNK), jnp.int32),
        pltpu.VMEM((2, _CHUNK), jnp.int32),
        pltpu.VMEM((2, _CHUNK), jnp.int32),
    ]
    if compute_counts:
        out_type.append(jax.ShapeDtypeStruct((_NPAD, 16), _F32))
        scratch.append(pltpu.VMEM((_CHUNK, 16), _F32))
    scratch.append(pltpu.VMEM_SHARED((_NPAD, _HD), _F32))
    if compute_counts:
        scratch.append(pltpu.VMEM_SHARED((_NPAD, 16), _F32))
    nsem = 9 if compute_counts else 7
    scratch.extend([pltpu.SemaphoreType.DMA] * nsem)
    return pl.kernel(
        functools.partial(_seg_sum_body, compute_counts),
        out_type=out_type,
        mesh=mesh,
        scratch_types=scratch,
        compiler_params=pltpu.CompilerParams(use_tc_tiling_on_sc=False),
    )


def _counts_body(dss, dsu, zcnt, ones_h, cnt_ss_o, cnt_su_o,
                 idxv, onesv, cntacc, sem):
    cid = lax.axis_index("c")
    sid = lax.axis_index("s")
    row0 = sid * _RSUB

    pltpu.sync_copy(zcnt.at[pl.ds(row0, _RSUB)], cntacc.at[pl.ds(row0, _RSUB)])
    pltpu.sync_copy(ones_h, onesv)

    def run(d_hbm, out_hbm):
        pltpu.sync_copy(d_hbm.at[sid], idxv)
        plsc.subcore_barrier()
        for j in range(4):
            pltpu.async_copy(onesv, cntacc.at[idxv.at[j]], sem, add=True)
        for j in range(4):
            pltpu.make_async_copy(onesv, cntacc.at[idxv.at[0]], sem).wait()
        plsc.subcore_barrier()
        pltpu.sync_copy(cntacc.at[pl.ds(row0, _RSUB)],
                        out_hbm.at[pl.ds(row0, _RSUB)])

    @pl.when(cid == 0)
    def _():
        run(dss, cnt_ss_o)

    @pl.when(cid == 1)
    def _():
        run(dsu, cnt_su_o)


_CCH = _ESUB // 4

_counts = pl.kernel(
    _counts_body,
    out_type=[jax.ShapeDtypeStruct((_NPAD, 16), _F32),
              jax.ShapeDtypeStruct((_NPAD, 16), _F32)],
    mesh=plsc.VectorSubcoreMesh(core_axis_name="c", subcore_axis_name="s"),
    scratch_types=[
        pltpu.VMEM((4, _CCH), jnp.int32),
        pltpu.VMEM((_CCH, 16), _F32),
        pltpu.VMEM_SHARED((_NPAD, 16), _F32),
        pltpu.SemaphoreType.DMA,
    ],
    compiler_params=pltpu.CompilerParams(use_tc_tiling_on_sc=False),
)


_seg_sum3 = _make_seg_sum3(False)


_R = 2000


def _dot(a, b):
    return jnp.dot(a, b, precision=_HP, preferred_element_type=_F32)


def _pre_body(xs, xu, w1r, b1, w2r, b2, pre1_o, pre2_o):
    pre1_o[...] = _dot(xs[...], w1r[...]) + b1[...]
    pre2_o[...] = _dot(xu[...], w2r[...]) + b2[...]


def _mid_item_body(agg1, cnt1, pre1, w1l, item0_o, item1_o):
    r1 = 1.0 / jnp.maximum(cnt1[:, 0:1], 1.0)
    w1l_m = w1l[...]
    item = (_dot(agg1[0] * r1, w1l_m[:_HD])
            + _dot(agg1[1] * r1, w1l_m[_HD:])
            + pre1[...])
    item = jnp.maximum(item, 0.0)
    item0_o[...] = item[:, :_HD]
    item1_o[...] = item[:, _HD:]


def _mid_user_body(agg2, cnt2, pre2, w2l, w3r, b3, u2r_o):
    r2 = 1.0 / jnp.maximum(cnt2[:, 0:1], 1.0)
    w2l_m = w2l[...]
    user2 = (_dot(agg2[0] * r2, w2l_m[:_HD])
             + _dot(agg2[1] * r2, w2l_m[_HD:])
             + pre2[...])
    user2 = jnp.maximum(user2, 0.0)
    u2r_o[...] = _dot(user2, w3r[...]) + b3[...]


def _fin_body(agg3, cnt2, u2r, w3l, wlin, blin, out_o):
    r = 1.0 / jnp.maximum(cnt2[:, 0:1], 1.0)
    w3l_m = w3l[...]
    u3 = (_dot(agg3[0] * r, w3l_m[:_HD])
          + _dot(agg3[1] * r, w3l_m[_HD:])
          + u2r[...])
    u3 = jnp.maximum(u3, 0.0)
    out_o[...] = _dot(u3, wlin[...]) + blin[...]


def _full(shape):
    return pl.BlockSpec(shape, lambda i: tuple(0 for _ in shape))


_rows3 = pl.BlockSpec((_NC, _R, _HD), lambda i: (0, i, 0))
_rowsD = pl.BlockSpec((_R, _D), lambda i: (i, 0))
_rowsC = pl.BlockSpec((_R, 16), lambda i: (i, 0))
_rowsH = pl.BlockSpec((_R, _HD), lambda i: (i, 0))

_pre = pl.pallas_call(
    _pre_body,
    grid=(_N // _R,),
    in_specs=[_rowsD, _rowsD,
              _full((_D, _D)), _full((1, _D)),
              _full((_D, _D)), _full((1, _D))],
    out_specs=[_rowsD, _rowsD],
    out_shape=[jax.ShapeDtypeStruct((_N, _D), _F32),
               jax.ShapeDtypeStruct((_N, _D), _F32)],
)

_mid_item = pl.pallas_call(
    _mid_item_body,
    grid=(_N // _R,),
    in_specs=[_rows3, _rowsC, _rowsD, _full((_D, _D))],
    out_specs=[_rowsH, _rowsH],
    out_shape=[jax.ShapeDtypeStruct((_N, _HD), _F32),
               jax.ShapeDtypeStruct((_N, _HD), _F32)],
)

_mid_user = pl.pallas_call(
    _mid_user_body,
    grid=(_N // _R,),
    in_specs=[_rows3, _rowsC, _rowsD,
              _full((_D, _D)), _full((_D, _D)), _full((1, _D))],
    out_specs=_rowsD,
    out_shape=jax.ShapeDtypeStruct((_N, _D), _F32),
)

_fin = pl.pallas_call(
    _fin_body,
    grid=(_N // _R,),
    in_specs=[_rows3, _rowsC, _rowsD,
              _full((_D, _D)), _full((_D, _O)), _full((1, _O))],
    out_specs=pl.BlockSpec((_R, _O), lambda i: (i, 0)),
    out_shape=jax.ShapeDtypeStruct((_N, _O), _F32),
)


def kernel(x_store, x_user, edge_index_store_store, edge_index_store_user,
           W1_l, b1, W1_r, W2_l, b2, W2_r, W3_l, b3, W3_r, W_lin, b_lin):
    xsl = x_store[:, :_HD]
    xsr = x_store[:, _HD:]
    ess3 = jnp.transpose(
        edge_index_store_store.reshape(2, _NS, _NSUP, 4, _CHUNK3),
        (1, 2, 3, 0, 4))
    esu3 = jnp.transpose(
        edge_index_store_user.reshape(2, _NS, _NSUP, 4, _CHUNK3),
        (1, 2, 3, 0, 4))
    dss = edge_index_store_store[1].reshape(_NS, 4, _CCH)
    dsu = edge_index_store_user[1].reshape(_NS, 4, _CCH)
    zrows = jnp.zeros((_NPAD, _HD), _F32)
    zcnt = jnp.zeros((_NPAD, 16), _F32)
    ones = jnp.ones((_CCH, 16), _F32)

    pre1, pre2 = _pre(x_store, x_user,
                      W1_r, b1.reshape(1, _D),
                      W2_r, b2.reshape(1, _D))
    cnt1, cnt2 = _counts(dss, dsu, zcnt, ones)
    (agg1,) = _seg_sum3(xsl, xsr, ess3, zrows)
    (agg2,) = _seg_sum3(xsl, xsr, esu3, zrows)
    item0, item1 = _mid_item(agg1, cnt1, pre1, W1_l)
    u2r = _mid_user(agg2, cnt2, pre2, W2_l, W3_r, b3.reshape(1, _D))
    (agg3,) = _seg_sum3(item0, item1, esu3, zrows)
    return _fin(agg3, cnt2, u2r, W3_l, W_lin, b_lin.reshape(1, _O))
